# Initial kernel scaffold; baseline (speedup 1.0000x reference)
import jax
import jax.numpy as jnp
from jax.experimental import pallas as pl


def kernel(x, edge_index, batch, edge_weight, params):
    raise NotImplementedError("write your pallas kernel here")


# speedup vs baseline: 250.4179x; 250.4179x over previous
import functools
import math

import jax
import jax.numpy as jnp
from jax import lax
from jax.experimental import pallas as pl
from jax.experimental.pallas import tpu as pltpu
from jax.experimental.pallas import tpu_sc as plsc

N = 10000
E = 320000
G = 50
NPG = 200
H = 128
C = 10
EPS = 1e-5
RATIO = 0.8

_KS = []
_npg = NPG
for _ in range(6):
    _KS.append(int(math.ceil(RATIO * _npg)))
    _npg = _KS[-1]

_BN_SCALE = 1.0 / math.sqrt(1.0 + EPS)
_NEG = -1e30


_GSQ = NPG * NPG
_PH = (13, 12)
_TRASH = 13 * _GSQ
_SPM = 564224
_DCH = 5000

_EPC = E // 16
_CH = 128
_NCH = (_EPC + _CH - 1) // _CH
_EPAD = _NCH * _CH

_ZCH = _SPM // 64


def _sc_body(src_hbm, dst_hbm, ew_hbm, out_hbm,
             src_v, dst_v, ew_v, idx_v, zb_v, ew_sh):
    c = lax.axis_index("c")
    s = lax.axis_index("s")

    ebase = s * _EPC
    pltpu.sync_copy(src_hbm.at[pl.ds(ebase, _EPC)], src_v.at[pl.ds(0, _EPC)])
    pltpu.sync_copy(dst_hbm.at[pl.ds(ebase, _EPC)], dst_v.at[pl.ds(0, _EPC)])
    pltpu.sync_copy(ew_hbm.at[pl.ds(ebase, _EPC)], ew_v.at[pl.ds(0, _EPC)])

    zero16 = jnp.zeros((16,), jnp.float32)
    lanes = lax.iota(jnp.int32, 16)

    for p in range(2):
        cnt = _PH[p]
        base = (c * 25 + p * _PH[0]) * _GSQ

        def zstore(i, _):
            zb_v[pl.ds(i * 16, 16)] = zero16
            return _

        lax.fori_loop(0, _ZCH // 16, zstore, 0, unroll=8)

        def zcopy(q, _):
            pltpu.sync_copy(
                zb_v, ew_sh.at[pl.ds(s * (_SPM // 16) + q * _ZCH, _ZCH)])
            return _

        lax.fori_loop(0, 4, zcopy, 0)

        gbase = c * 25 + p * _PH[0]

        def compute(i, _):
            o = i * 16
            sv = src_v[pl.ds(o, 16)]
            dv = dst_v[pl.ds(o, 16)]
            g = lax.div(dv, jnp.int32(NPG))
            fl = dv * NPG + sv - g * NPG
            loc = fl - gbase * _GSQ
            pos = o + lanes
            ok = (pos < _EPC) & (g >= gbase) & (g < gbase + cnt)
            trash = _TRASH + ((o + lanes) & 4095)
            idx_v[i // (_CH // 16), pl.ds((i % (_CH // 16)) * 16, 16)] = (
                jnp.where(ok, loc, trash))
            return _

        lax.fori_loop(0, _EPAD // 16, compute, 0, unroll=4)

        plsc.subcore_barrier()

        def scatter(j, _):
            pltpu.sync_copy(ew_v.at[pl.ds(j * _CH, _CH)],
                            ew_sh.at[idx_v.at[j]], add=True)
            return _

        lax.fori_loop(0, _NCH, scatter, 0)

        plsc.subcore_barrier()

        cnt_pad = cnt + (cnt % 2)
        nch = cnt_pad * _GSQ // _DCH // 16

        def drain(q, _):
            off = (q * 16 + s) * _DCH
            pltpu.sync_copy(ew_sh.at[pl.ds(off, _DCH)],
                            zb_v.at[pl.ds(0, _DCH)])
            pltpu.sync_copy(zb_v.at[pl.ds(0, _DCH)],
                            out_hbm.at[pl.ds(base + off, _DCH)])
            return _

        lax.fori_loop(0, nch, drain, 0)

        plsc.subcore_barrier()


def _build_ew(edge_index, edge_weight):
    mesh = plsc.VectorSubcoreMesh(core_axis_name="c", subcore_axis_name="s")
    k = pl.kernel(
        _sc_body,
        mesh=mesh,
        out_type=jax.ShapeDtypeStruct((G * _GSQ,), jnp.float32),
        scratch_types=[
            pltpu.VMEM((_EPAD,), jnp.int32),
            pltpu.VMEM((_EPAD,), jnp.int32),
            pltpu.VMEM((_EPAD,), jnp.float32),
            pltpu.VMEM((_NCH, _CH), jnp.int32),
            pltpu.VMEM((_ZCH,), jnp.float32),
            pltpu.VMEM_SHARED((_SPM,), jnp.float32),
        ],
    )
    return k(edge_index[0], edge_index[1], edge_weight).reshape(G, NPG, NPG)


def _tc_body(x_ref, ew_ref, w_ref, b_ref, bng_ref, bnb_ref, pw_ref,
             d1w_ref, d1b_ref, d2w_ref, d2b_ref, out_ref):
    h = x_ref[...]
    ew = ew_ref[0]
    ii = lax.broadcasted_iota(jnp.int32, (NPG, NPG), 0)
    jj = lax.broadcasted_iota(jnp.int32, (NPG, NPG), 1)
    eye = (ii == jj).astype(jnp.float32)

    def rowvec(col):
        return jnp.sum(col * eye, axis=0, keepdims=True)

    alive = jnp.ones((NPG, 1), jnp.float32)
    flat_acc = jnp.zeros((1, 2 * H), jnp.float32)

    for i in range(6):
        W = w_ref[i]
        xw = jnp.dot(h, W, preferred_element_type=jnp.float32)
        deg = jnp.sum(ew, axis=1, keepdims=True) + alive
        dis = jnp.where(deg > 0, lax.rsqrt(deg), 0.0)
        dis_r = rowvec(dis)
        a = ew * dis * dis_r
        conv = (jnp.dot(a, xw, preferred_element_type=jnp.float32)
                + dis * dis * xw + b_ref[i])
        hr = jnp.maximum(conv * _BN_SCALE * bng_ref[i] + bnb_ref[i], 0.0)
        pw = pw_ref[i]
        ss = jnp.sum(pw * pw, axis=1, keepdims=True)
        z = jnp.sum(hr * pw, axis=1, keepdims=True) / jnp.sqrt(ss)
        score = jnp.tanh(z)
        sc = jnp.where(alive > 0, score, _NEG)
        sc_r = rowvec(sc)
        gt = (sc_r > sc).astype(jnp.float32)
        eqlt = jnp.where((sc_r == sc) & (ii > jj), 1.0, 0.0)
        rank = jnp.sum(gt + eqlt, axis=1, keepdims=True)
        k = _KS[i]
        sel = jnp.where((rank < k) & (alive > 0), 1.0, 0.0)
        hn = hr * score
        mean = jnp.sum(sel * hn, axis=0, keepdims=True) / k
        mx = jnp.max(jnp.where(sel > 0, hn, _NEG), axis=0, keepdims=True)
        flat_acc = flat_acc + jnp.concatenate([mean, mx], axis=1)
        ew = ew * sel * rowvec(sel)
        alive = sel
        h = hn

    hd = jnp.maximum(
        jnp.dot(flat_acc, d1w_ref[...], preferred_element_type=jnp.float32)
        + d1b_ref[...], 0.0)
    res = (jnp.dot(hd, d2w_ref[...], preferred_element_type=jnp.float32)
           + d2b_ref[...])
    out_ref[...] = res[None]


def _run_tc(x, ew, params, interpret=False):
    ws = jnp.stack([params['conv%d_W' % i] for i in range(6)])
    bs = jnp.stack([params['conv%d_b' % i] for i in range(6)])[:, None, :]
    bngs = jnp.stack([params['bn%d_g' % i] for i in range(6)])[:, None, :]
    bnbs = jnp.stack([params['bn%d_b' % i] for i in range(6)])[:, None, :]
    pws = jnp.stack([params['pool%d_w' % i] for i in range(6)])[:, None, :]
    cst = lambda *_: tuple(0 for _ in range(len(_)))
    grid = (G,)
    return pl.pallas_call(
        _tc_body,
        grid=grid,
        in_specs=[
            pl.BlockSpec((NPG, H), lambda g: (g, 0)),
            pl.BlockSpec((1, NPG, NPG), lambda g: (g, 0, 0)),
            pl.BlockSpec((6, H, H), lambda g: (0, 0, 0)),
            pl.BlockSpec((6, 1, H), lambda g: (0, 0, 0)),
            pl.BlockSpec((6, 1, H), lambda g: (0, 0, 0)),
            pl.BlockSpec((6, 1, H), lambda g: (0, 0, 0)),
            pl.BlockSpec((6, 1, H), lambda g: (0, 0, 0)),
            pl.BlockSpec((2 * H, 4 * H), lambda g: (0, 0)),
            pl.BlockSpec((1, 4 * H), lambda g: (0, 0)),
            pl.BlockSpec((4 * H, C), lambda g: (0, 0)),
            pl.BlockSpec((1, C), lambda g: (0, 0)),
        ],
        out_specs=pl.BlockSpec((1, 1, C), lambda g: (g, 0, 0)),
        out_shape=jax.ShapeDtypeStruct((G, 1, C), jnp.float32),
        interpret=interpret,
    )(x, ew, ws, bs, bngs, bnbs, pws,
      params['d1_W'], params['d1_b'][None, :],
      params['d2_W'], params['d2_b'][None, :]).reshape(G, C)


def kernel(x, edge_index, batch, edge_weight, params):
    ew = _build_ew(edge_index, edge_weight)
    return _run_tc(x, ew, params)


# speedup vs baseline: 262.8334x; 1.0496x over previous
import math

import jax
import jax.numpy as jnp
from jax import lax
from jax.experimental import pallas as pl
from jax.experimental.pallas import tpu as pltpu
from jax.experimental.pallas import tpu_sc as plsc

N = 10000
E = 320000
G = 50
NPG = 200
H = 128
C = 10
EPS = 1e-5
RATIO = 0.8

_KS = []
_npg = NPG
for _ in range(6):
    _KS.append(int(math.ceil(RATIO * _npg)))
    _npg = _KS[-1]

_GPP = 2
_BN_SCALE = 1.0 / math.sqrt(1.0 + EPS)
_NEG = -1e30


_GSQ = NPG * NPG
_PH = (13, 12)
_TRASH = 13 * _GSQ
_SPM = 564224
_DCH = 5000

_EPC = E // 16
_CH = 128
_FIRE = 8
_NCH = (_EPC + _CH * _FIRE - 1) // (_CH * _FIRE) * _FIRE
_EPAD = _NCH * _CH

_ZCH = _SPM // 64


def _sc_body(src_hbm, dst_hbm, ew_hbm, out_hbm,
             src_v, dst_v, ew_v, idx_v, zb_v, ew_sh, sem):
    c = lax.axis_index("c")
    s = lax.axis_index("s")

    ebase = s * _EPC
    pltpu.sync_copy(src_hbm.at[pl.ds(ebase, _EPC)], src_v.at[pl.ds(0, _EPC)])
    pltpu.sync_copy(dst_hbm.at[pl.ds(ebase, _EPC)], dst_v.at[pl.ds(0, _EPC)])
    pltpu.sync_copy(ew_hbm.at[pl.ds(ebase, _EPC)], ew_v.at[pl.ds(0, _EPC)])

    zero16 = jnp.zeros((16,), jnp.float32)
    lanes = lax.iota(jnp.int32, 16)

    for p in range(2):
        cnt = _PH[p]
        base = (c * 25 + p * _PH[0]) * _GSQ

        def zstore(i, _):
            zb_v[pl.ds(i * 16, 16)] = zero16
            return _

        lax.fori_loop(0, _ZCH // 16, zstore, 0, unroll=8)

        def zcopy(q, _):
            pltpu.sync_copy(
                zb_v, ew_sh.at[pl.ds(s * (_SPM // 16) + q * _ZCH, _ZCH)])
            return _

        lax.fori_loop(0, 4, zcopy, 0)

        gbase = c * 25 + p * _PH[0]

        def compute(i, _):
            o = i * 16
            sv = src_v[pl.ds(o, 16)]
            dv = dst_v[pl.ds(o, 16)]
            g = lax.div(dv, jnp.int32(NPG))
            fl = dv * NPG + sv - g * NPG
            loc = fl - gbase * _GSQ
            pos = o + lanes
            ok = (pos < _EPC) & (g >= gbase) & (g < gbase + cnt)
            trash = _TRASH + ((o + lanes) & 4095)
            idx_v[i // (_CH // 16), pl.ds((i % (_CH // 16)) * 16, 16)] = (
                jnp.where(ok, loc, trash))
            return _

        lax.fori_loop(0, _EPAD // 16, compute, 0, unroll=4)

        plsc.subcore_barrier()

        def scatter(jg, _):
            for t in range(_FIRE):
                jj = jg * _FIRE + t
                pltpu.async_copy(ew_v.at[pl.ds(jj * _CH, _CH)],
                                 ew_sh.at[idx_v.at[jj]], sem, add=True)
            for t in range(_FIRE):
                jj = jg * _FIRE + t
                pltpu.make_async_copy(ew_v.at[pl.ds(jj * _CH, _CH)],
                                      ew_sh.at[idx_v.at[jj]], sem).wait()
            return _

        lax.fori_loop(0, _NCH // _FIRE, scatter, 0)

        plsc.subcore_barrier()

        cnt_pad = cnt + (cnt % 2)
        nch = cnt_pad * _GSQ // _DCH // 16

        def drain(q, _):
            off = (q * 16 + s) * _DCH
            pltpu.sync_copy(ew_sh.at[pl.ds(off, _DCH)],
                            zb_v.at[pl.ds(0, _DCH)])
            pltpu.sync_copy(zb_v.at[pl.ds(0, _DCH)],
                            out_hbm.at[pl.ds(base + off, _DCH)])
            return _

        lax.fori_loop(0, nch, drain, 0)

        plsc.subcore_barrier()


def _build_ew(edge_index, edge_weight):
    mesh = plsc.VectorSubcoreMesh(core_axis_name="c", subcore_axis_name="s")
    k = pl.kernel(
        _sc_body,
        mesh=mesh,
        out_type=jax.ShapeDtypeStruct((G * _GSQ,), jnp.float32),
        scratch_types=[
            pltpu.VMEM((_EPAD,), jnp.int32),
            pltpu.VMEM((_EPAD,), jnp.int32),
            pltpu.VMEM((_EPAD,), jnp.float32),
            pltpu.VMEM((_NCH, _CH), jnp.int32),
            pltpu.VMEM((_ZCH,), jnp.float32),
            pltpu.VMEM_SHARED((_SPM,), jnp.float32),
            pltpu.SemaphoreType.DMA,
        ],
    )
    return k(edge_index[0], edge_index[1], edge_weight).reshape(G, NPG, NPG)


def _tc_body(x_ref, ew_ref, w_ref, b_ref, bng_ref, bnb_ref, pw_ref,
             d1w_ref, d1b_ref, d2w_ref, d2b_ref, out_ref):
    ii = lax.broadcasted_iota(jnp.int32, (NPG, NPG), 0)
    jj = lax.broadcasted_iota(jnp.int32, (NPG, NPG), 1)
    eye = (ii == jj).astype(jnp.float32)

    def rowvec(col):
        return jnp.sum(col * eye, axis=0, keepdims=True)

    for sub in range(_GPP):
        _one_graph(x_ref[pl.ds(sub * NPG, NPG), :], ew_ref[sub], w_ref, b_ref,
                   bng_ref, bnb_ref, pw_ref, d1w_ref, d1b_ref, d2w_ref,
                   d2b_ref, out_ref, sub, eye, rowvec, ii, jj)


def _one_graph(h, ew, w_ref, b_ref, bng_ref, bnb_ref, pw_ref,
               d1w_ref, d1b_ref, d2w_ref, d2b_ref, out_ref, sub,
               eye, rowvec, ii, jj):
    alive = jnp.ones((NPG, 1), jnp.float32)
    flat_acc = jnp.zeros((1, 2 * H), jnp.float32)

    for i in range(6):
        W = w_ref[i]
        xw = jnp.dot(h, W, preferred_element_type=jnp.float32)
        deg = jnp.sum(ew, axis=1, keepdims=True) + alive
        dis = jnp.where(deg > 0, lax.rsqrt(deg), 0.0)
        dis_r = rowvec(dis)
        a = ew * dis * dis_r
        conv = (jnp.dot(a, xw, preferred_element_type=jnp.float32)
                + dis * dis * xw + b_ref[i])
        hr = jnp.maximum(conv * _BN_SCALE * bng_ref[i] + bnb_ref[i], 0.0)
        pw = pw_ref[i]
        ss = jnp.sum(pw * pw, axis=1, keepdims=True)
        z = jnp.sum(hr * pw, axis=1, keepdims=True) / jnp.sqrt(ss)
        score = jnp.tanh(z)
        sc = jnp.where(alive > 0, score, _NEG)
        sc_r = rowvec(sc)
        gt = (sc_r > sc).astype(jnp.float32)
        eqlt = jnp.where((sc_r == sc) & (ii > jj), 1.0, 0.0)
        rank = jnp.sum(gt + eqlt, axis=1, keepdims=True)
        k = _KS[i]
        sel = jnp.where((rank < k) & (alive > 0), 1.0, 0.0)
        hn = hr * score
        mean = jnp.sum(sel * hn, axis=0, keepdims=True) / k
        mx = jnp.max(jnp.where(sel > 0, hn, _NEG), axis=0, keepdims=True)
        flat_acc = flat_acc + jnp.concatenate([mean, mx], axis=1)
        ew = ew * sel * rowvec(sel)
        alive = sel
        h = hn

    hd = jnp.maximum(
        jnp.dot(flat_acc, d1w_ref[...], preferred_element_type=jnp.float32)
        + d1b_ref[...], 0.0)
    res = (jnp.dot(hd, d2w_ref[...], preferred_element_type=jnp.float32)
           + d2b_ref[...])
    out_ref[sub] = res


def _run_tc(x, ew, params, interpret=False):
    ws = jnp.stack([params['conv%d_W' % i] for i in range(6)])
    bs = jnp.stack([params['conv%d_b' % i] for i in range(6)])[:, None, :]
    bngs = jnp.stack([params['bn%d_g' % i] for i in range(6)])[:, None, :]
    bnbs = jnp.stack([params['bn%d_b' % i] for i in range(6)])[:, None, :]
    pws = jnp.stack([params['pool%d_w' % i] for i in range(6)])[:, None, :]
    grid = (G // _GPP,)
    return pl.pallas_call(
        _tc_body,
        grid=grid,
        in_specs=[
            pl.BlockSpec((_GPP * NPG, H), lambda g: (g, 0)),
            pl.BlockSpec((_GPP, NPG, NPG), lambda g: (g, 0, 0)),
            pl.BlockSpec((6, H, H), lambda g: (0, 0, 0)),
            pl.BlockSpec((6, 1, H), lambda g: (0, 0, 0)),
            pl.BlockSpec((6, 1, H), lambda g: (0, 0, 0)),
            pl.BlockSpec((6, 1, H), lambda g: (0, 0, 0)),
            pl.BlockSpec((6, 1, H), lambda g: (0, 0, 0)),
            pl.BlockSpec((2 * H, 4 * H), lambda g: (0, 0)),
            pl.BlockSpec((1, 4 * H), lambda g: (0, 0)),
            pl.BlockSpec((4 * H, C), lambda g: (0, 0)),
            pl.BlockSpec((1, C), lambda g: (0, 0)),
        ],
        out_specs=pl.BlockSpec((_GPP, 1, C), lambda g: (g, 0, 0)),
        out_shape=jax.ShapeDtypeStruct((G, 1, C), jnp.float32),
        interpret=interpret,
    )(x, ew, ws, bs, bngs, bnbs, pws,
      params['d1_W'], params['d1_b'][None, :],
      params['d2_W'], params['d2_b'][None, :]).reshape(G, C)


def kernel(x, edge_index, batch, edge_weight, params):
    ew = _build_ew(edge_index, edge_weight)
    return _run_tc(x, ew, params)


# speedup vs baseline: 382.9414x; 1.4570x over previous
import math

import jax
import jax.numpy as jnp
from jax import lax
from jax.experimental import pallas as pl
from jax.experimental.pallas import tpu as pltpu
from jax.experimental.pallas import tpu_sc as plsc

N = 10000
E = 320000
G = 50
NPG = 200
H = 128
C = 10
EPS = 1e-5
RATIO = 0.8

_KS = []
_npg = NPG
for _ in range(6):
    _KS.append(int(math.ceil(RATIO * _npg)))
    _npg = _KS[-1]

_GPP = 10
_BN_SCALE = 1.0 / math.sqrt(1.0 + EPS)
_NEG = -1e30


_GSQ = NPG * NPG
_PH = (13, 12)
_TRASH = 13 * _GSQ
_SPM = 562176
_DCH = 5000

_EPC = E // 16
_CH = 128
_FIRE = 8
_NCH = (_EPC + _CH * _FIRE - 1) // (_CH * _FIRE) * _FIRE
_EPAD = _NCH * _CH

_ZCH = _SPM // 64


def _sc_body(src_hbm, dst_hbm, ew_hbm, out_hbm,
             src_v, dst_v, ew_v, idx_v, zb_v, db_v, ew_sh, sem):
    c = lax.axis_index("c")
    s = lax.axis_index("s")

    ebase = s * _EPC
    pltpu.async_copy(src_hbm.at[pl.ds(ebase, _EPC)], src_v.at[pl.ds(0, _EPC)],
                     sem)
    pltpu.async_copy(dst_hbm.at[pl.ds(ebase, _EPC)], dst_v.at[pl.ds(0, _EPC)],
                     sem)
    pltpu.async_copy(ew_hbm.at[pl.ds(ebase, _EPC)], ew_v.at[pl.ds(0, _EPC)],
                     sem)

    zero16 = jnp.zeros((16,), jnp.float32)
    lanes = lax.iota(jnp.int32, 16)

    def zstore(i, _):
        zb_v[pl.ds(i * 16, 16)] = zero16
        return _

    lax.fori_loop(0, _ZCH // 16, zstore, 0, unroll=8)

    pltpu.make_async_copy(src_hbm.at[pl.ds(ebase, _EPC)],
                          src_v.at[pl.ds(0, _EPC)], sem).wait()
    pltpu.make_async_copy(dst_hbm.at[pl.ds(ebase, _EPC)],
                          dst_v.at[pl.ds(0, _EPC)], sem).wait()
    pltpu.make_async_copy(ew_hbm.at[pl.ds(ebase, _EPC)],
                          ew_v.at[pl.ds(0, _EPC)], sem).wait()

    for p in range(2):
        cnt = _PH[p]
        base = (c * 25 + p * _PH[0]) * _GSQ

        for q in range(4):
            pltpu.async_copy(
                zb_v, ew_sh.at[pl.ds(s * (_SPM // 16) + q * _ZCH, _ZCH)], sem)

        gbase = c * 25 + p * _PH[0]

        def compute(i, _):
            o = i * 16
            sv = src_v[pl.ds(o, 16)]
            dv = dst_v[pl.ds(o, 16)]
            g = lax.div(dv, jnp.int32(NPG))
            fl = dv * NPG + sv - g * NPG
            loc = fl - gbase * _GSQ
            pos = o + lanes
            ok = (pos < _EPC) & (g >= gbase) & (g < gbase + cnt)
            trash = _TRASH + ((o + lanes) & 2047)
            idx_v[i // (_CH // 16), pl.ds((i % (_CH // 16)) * 16, 16)] = (
                jnp.where(ok, loc, trash))
            return _

        lax.fori_loop(0, _EPAD // 16, compute, 0, unroll=4)

        for q in range(4):
            pltpu.make_async_copy(
                zb_v, ew_sh.at[pl.ds(s * (_SPM // 16) + q * _ZCH, _ZCH)],
                sem).wait()

        plsc.subcore_barrier()

        def scatter(jg, _):
            for t in range(_FIRE):
                jj = jg * _FIRE + t
                pltpu.async_copy(ew_v.at[pl.ds(jj * _CH, _CH)],
                                 ew_sh.at[idx_v.at[jj]], sem, add=True)
            for t in range(_FIRE):
                jj = jg * _FIRE + t
                pltpu.make_async_copy(ew_v.at[pl.ds(jj * _CH, _CH)],
                                      ew_sh.at[idx_v.at[jj]], sem).wait()
            return _

        lax.fori_loop(0, _NCH // _FIRE, scatter, 0)

        plsc.subcore_barrier()

        cnt_pad = cnt + (cnt % 2)
        nch = cnt_pad * _GSQ // _DCH // 16

        def drain(q, _):
            off = (q * 16 + s) * _DCH
            pltpu.sync_copy(ew_sh.at[pl.ds(off, _DCH)],
                            db_v.at[pl.ds(0, _DCH)])
            pltpu.sync_copy(db_v.at[pl.ds(0, _DCH)],
                            out_hbm.at[pl.ds(base + off, _DCH)])
            return _

        lax.fori_loop(0, nch, drain, 0)

        plsc.subcore_barrier()


def _build_ew(edge_index, edge_weight):
    mesh = plsc.VectorSubcoreMesh(core_axis_name="c", subcore_axis_name="s")
    k = pl.kernel(
        _sc_body,
        mesh=mesh,
        out_type=jax.ShapeDtypeStruct((G * _GSQ,), jnp.float32),
        scratch_types=[
            pltpu.VMEM((_EPAD,), jnp.int32),
            pltpu.VMEM((_EPAD,), jnp.int32),
            pltpu.VMEM((_EPAD,), jnp.float32),
            pltpu.VMEM((_NCH, _CH), jnp.int32),
            pltpu.VMEM((_ZCH,), jnp.float32),
            pltpu.VMEM((_DCH,), jnp.float32),
            pltpu.VMEM_SHARED((_SPM,), jnp.float32),
            pltpu.SemaphoreType.DMA,
        ],
    )
    return k(edge_index[0], edge_index[1], edge_weight).reshape(G, NPG, NPG)


def _tc_body(x_ref, ew_ref, w_ref, b_ref, bng_ref, bnb_ref, pw_ref,
             d1w_ref, d1b_ref, d2w_ref, d2b_ref, out_ref):
    ii = lax.broadcasted_iota(jnp.int32, (NPG, NPG), 0)
    jj = lax.broadcasted_iota(jnp.int32, (NPG, NPG), 1)
    eye = (ii == jj).astype(jnp.float32)

    def rowvec(col):
        return jnp.sum(col * eye, axis=0, keepdims=True)

    hs = [x_ref[pl.ds(sub * NPG, NPG), :] for sub in range(_GPP)]
    ews = [ew_ref[sub] for sub in range(_GPP)]
    alives = [jnp.ones((NPG, 1), jnp.float32) for _ in range(_GPP)]
    flats = [jnp.zeros((1, 2 * H), jnp.float32) for _ in range(_GPP)]

    for i in range(6):
        k = _KS[i]
        W = w_ref[i]
        pw = pw_ref[i]
        ss = jnp.sum(pw * pw, axis=1, keepdims=True)
        for sub in range(_GPP):
            h, ew, alive = hs[sub], ews[sub], alives[sub]
            xw = jnp.dot(h, W, preferred_element_type=jnp.float32)
            deg = jnp.sum(ew, axis=1, keepdims=True) + alive
            dis = jnp.where(deg > 0, lax.rsqrt(deg), 0.0)
            y = dis * xw
            conv = (dis * jnp.dot(ew, y, preferred_element_type=jnp.float32)
                    + dis * y + b_ref[i])
            hr = jnp.maximum(conv * _BN_SCALE * bng_ref[i] + bnb_ref[i], 0.0)
            z = jnp.sum(hr * pw, axis=1, keepdims=True) / jnp.sqrt(ss)
            score = jnp.tanh(z)
            sc = jnp.where(alive > 0, score, _NEG)
            sc_r = rowvec(sc)
            gt = (sc_r > sc) | ((sc_r == sc) & (ii > jj))
            rank = jnp.sum(jnp.where(gt, 1.0, 0.0), axis=1, keepdims=True)
            sel = jnp.where((rank < k) & (alive > 0), 1.0, 0.0)
            hn = hr * score
            mean = jnp.sum(sel * hn, axis=0, keepdims=True) / k
            mx = jnp.max(jnp.where(sel > 0, hn, _NEG), axis=0, keepdims=True)
            flats[sub] = flats[sub] + jnp.concatenate([mean, mx], axis=1)
            ews[sub] = ew * rowvec(sel)
            alives[sub] = sel
            hs[sub] = hn

    for sub in range(_GPP):
        hd = jnp.maximum(
            jnp.dot(flats[sub], d1w_ref[...], preferred_element_type=jnp.float32)
            + d1b_ref[...], 0.0)
        res = (jnp.dot(hd, d2w_ref[...], preferred_element_type=jnp.float32)
               + d2b_ref[...])
        out_ref[sub] = res


def _run_tc(x, ew, params, interpret=False):
    ws = jnp.stack([params['conv%d_W' % i] for i in range(6)])
    bs = jnp.stack([params['conv%d_b' % i] for i in range(6)])[:, None, :]
    bngs = jnp.stack([params['bn%d_g' % i] for i in range(6)])[:, None, :]
    bnbs = jnp.stack([params['bn%d_b' % i] for i in range(6)])[:, None, :]
    pws = jnp.stack([params['pool%d_w' % i] for i in range(6)])[:, None, :]
    grid = (G // _GPP,)
    return pl.pallas_call(
        _tc_body,
        grid=grid,
        in_specs=[
            pl.BlockSpec((_GPP * NPG, H), lambda g: (g, 0)),
            pl.BlockSpec((_GPP, NPG, NPG), lambda g: (g, 0, 0)),
            pl.BlockSpec((6, H, H), lambda g: (0, 0, 0)),
            pl.BlockSpec((6, 1, H), lambda g: (0, 0, 0)),
            pl.BlockSpec((6, 1, H), lambda g: (0, 0, 0)),
            pl.BlockSpec((6, 1, H), lambda g: (0, 0, 0)),
            pl.BlockSpec((6, 1, H), lambda g: (0, 0, 0)),
            pl.BlockSpec((2 * H, 4 * H), lambda g: (0, 0)),
            pl.BlockSpec((1, 4 * H), lambda g: (0, 0)),
            pl.BlockSpec((4 * H, C), lambda g: (0, 0)),
            pl.BlockSpec((1, C), lambda g: (0, 0)),
        ],
        out_specs=pl.BlockSpec((_GPP, 1, C), lambda g: (g, 0, 0)),
        out_shape=jax.ShapeDtypeStruct((G, 1, C), jnp.float32),
        interpret=interpret,
    )(x, ew, ws, bs, bngs, bnbs, pws,
      params['d1_W'], params['d1_b'][None, :],
      params['d2_W'], params['d2_b'][None, :]).reshape(G, C)


def kernel(x, edge_index, batch, edge_weight, params):
    ew = _build_ew(edge_index, edge_weight)
    return _run_tc(x, ew, params)
